# baseline (speedup 1.0000x reference)
import jax
import jax.numpy as jnp
from jax import lax
from jax.experimental import pallas as pl
from jax.experimental.pallas import tpu as pltpu
from jax.experimental.pallas import tpu_sc as plsc

B = 4096
LP = 200
LQ = 20
LC = 20
DW = 300
DP = 12
DN = 8
DR = 10

NC = 2
NS = 16
NW = NC * NS
CH = 128

_TASKS = (
    ("p", "W_word", DW, B * LP),
    ("q", "W_word", DW, B * LQ),
    ("c", "W_word", DW, B * LC),
    ("pPos", "W_pos", DP, B * LP),
    ("pNer", "W_ner", DN, B * LP),
    ("qPos", "W_pos", DP, B * LQ),
    ("pQRel", "W_rel", DR, B * LP),
    ("pCRel", "W_rel", DR, B * LP),
)


def _sc_body(p, q, c, pPos, pNer, qPos, pQRel, pCRel,
             W_word, W_pos, W_ner, W_rel,
             o0, o1, o2, o3, o4, o5, o6, o7,
             idx_v, rows_w, rows_p, rows_n, rows_r, sem):
    wid = lax.axis_index("s") * NC + lax.axis_index("c")
    idx_refs = {"p": p, "q": q, "c": c, "pPos": pPos, "pNer": pNer,
                "qPos": qPos, "pQRel": pQRel, "pCRel": pCRel}
    tab_refs = {"W_word": W_word, "W_pos": W_pos, "W_ner": W_ner,
                "W_rel": W_rel}
    outs = (o0, o1, o2, o3, o4, o5, o6, o7)
    rows_bufs = {DW: rows_w, DP: rows_p, DN: rows_n, DR: rows_r}

    for t, (iname, tname, d, nrows) in enumerate(_TASKS):
        idx_hbm = idx_refs[iname]
        tab_hbm = tab_refs[tname]
        out_hbm = outs[t]
        rows_v = rows_bufs[d]
        per_w = nrows // NW
        nch = per_w // CH
        base = wid * per_w

        @pl.loop(0, nch)
        def _chunk(g, base=base, idx_hbm=idx_hbm, tab_hbm=tab_hbm,
                   out_hbm=out_hbm, rows_v=rows_v):
            r0 = base + g * CH
            pltpu.sync_copy(idx_hbm.at[pl.ds(r0, CH)], idx_v)
            pltpu.async_copy(tab_hbm.at[idx_v], rows_v, sem).wait()
            pltpu.sync_copy(rows_v, out_hbm.at[pl.ds(r0, CH)])


@jax.jit
def _sc_gather(p, q, c, pPos, pNer, qPos, pQRel, pCRel,
               W_word, W_pos, W_ner, W_rel):
    out_type = [
        jax.ShapeDtypeStruct((B * LP, 4 * DW), jnp.uint8),
        jax.ShapeDtypeStruct((B * LQ, 4 * DW), jnp.uint8),
        jax.ShapeDtypeStruct((B * LC, 4 * DW), jnp.uint8),
        jax.ShapeDtypeStruct((B * LP, 4 * DP), jnp.uint8),
        jax.ShapeDtypeStruct((B * LP, 4 * DN), jnp.uint8),
        jax.ShapeDtypeStruct((B * LQ, 4 * DP), jnp.uint8),
        jax.ShapeDtypeStruct((B * LP, 4 * DR), jnp.uint8),
        jax.ShapeDtypeStruct((B * LP, 4 * DR), jnp.uint8),
    ]
    scratch = [
        pltpu.VMEM((CH,), jnp.int32),
        pltpu.VMEM((CH, 4 * DW), jnp.uint8),
        pltpu.VMEM((CH, 4 * DP), jnp.uint8),
        pltpu.VMEM((CH, 4 * DN), jnp.uint8),
        pltpu.VMEM((CH, 4 * DR), jnp.uint8),
        pltpu.SemaphoreType.DMA,
    ]
    mesh = plsc.VectorSubcoreMesh(core_axis_name="c", subcore_axis_name="s")
    fn = pl.kernel(_sc_body, out_type=out_type, mesh=mesh,
                   scratch_types=scratch,
                   compiler_params=pltpu.CompilerParams(
                       use_tc_tiling_on_sc=False))
    return fn(p, q, c, pPos, pNer, qPos, pQRel, pCRel,
              W_word, W_pos, W_ner, W_rel)


def _as_bytes(w):
    v, d = w.shape
    return lax.bitcast_convert_type(w, jnp.uint8).reshape(v, 4 * d)


def _as_f32(o, b, l, d):
    return lax.bitcast_convert_type(
        o.reshape(b, l, d, 4), jnp.float32)


def kernel(p, q, c, pPos, pNer, qPos, pQRel, pCRel,
           W_word, W_pos, W_ner, W_rel):
    flat = [a.reshape(-1).astype(jnp.int32)
            for a in (p, q, c, pPos, pNer, qPos, pQRel, pCRel)]
    outs = _sc_gather(*flat, _as_bytes(W_word), _as_bytes(W_pos),
                      _as_bytes(W_ner), _as_bytes(W_rel))
    return (
        _as_f32(outs[0], B, LP, DW),
        _as_f32(outs[1], B, LQ, DW),
        _as_f32(outs[2], B, LC, DW),
        _as_f32(outs[3], B, LP, DP),
        _as_f32(outs[4], B, LP, DN),
        _as_f32(outs[5], B, LQ, DP),
        _as_f32(outs[6], B, LP, DR),
        _as_f32(outs[7], B, LP, DR),
    )


# speedup vs baseline: 4.0106x; 4.0106x over previous
import jax
import jax.numpy as jnp
from jax import lax
from jax.experimental import pallas as pl
from jax.experimental.pallas import tpu as pltpu
from jax.experimental.pallas import tpu_sc as plsc

B = 4096
LP = 200
LQ = 20
LC = 20
DW = 300
DP = 12
DN = 8
DR = 10

NC = 2
NS = 16
NW = NC * NS
CH = 64


def _pad8(d):
    return (d + 7) // 8 * 8


_TASKS = (
    ("p", "W_word", DW, B * LP),
    ("q", "W_word", DW, B * LQ),
    ("c", "W_word", DW, B * LC),
    ("pPos", "W_pos", DP, B * LP),
    ("pNer", "W_ner", DN, B * LP),
    ("qPos", "W_pos", DP, B * LQ),
    ("pQRel", "W_rel", DR, B * LP),
    ("pCRel", "W_rel", DR, B * LP),
)


def _sc_body(p, q, c, pPos, pNer, qPos, pQRel, pCRel,
             W_word, W_pos, W_ner, W_rel,
             o0, o1, o2, o3, o4, o5, o6, o7,
             idx_v, rows_w, rows_p, rows_n, rows_r,
             pk_w, pk_p, pk_r, sem):
    wid = lax.axis_index("s") * NC + lax.axis_index("c")
    idx_refs = {"p": p, "q": q, "c": c, "pPos": pPos, "pNer": pNer,
                "qPos": qPos, "pQRel": pQRel, "pCRel": pCRel}
    tab_refs = {"W_word": W_word, "W_pos": W_pos, "W_ner": W_ner,
                "W_rel": W_rel}
    outs = (o0, o1, o2, o3, o4, o5, o6, o7)
    rows_bufs = {DW: rows_w, DP: rows_p, DN: rows_n, DR: rows_r}
    pk_bufs = {DW: pk_w, DP: pk_p, DR: pk_r}

    for t, (iname, tname, d, nrows) in enumerate(_TASKS):
        idx_hbm = idx_refs[iname]
        tab_hbm = tab_refs[tname]
        out_hbm = outs[t]
        rows_v = rows_bufs[d]
        per_w = nrows // NW
        nch = per_w // CH
        base = wid * per_w

        if d == DN:
            @pl.loop(0, nch)
            def _chunk8(g, base=base, idx_hbm=idx_hbm, tab_hbm=tab_hbm,
                        out_hbm=out_hbm, rows_v=rows_v):
                r0 = base + g * CH
                pltpu.sync_copy(idx_hbm.at[pl.ds(r0, CH)], idx_v)
                pltpu.async_copy(tab_hbm.at[idx_v], rows_v, sem).wait()
                pltpu.sync_copy(rows_v, out_hbm.at[pl.ds(r0, CH)])
            continue

        pk_v = pk_bufs[d]
        nvr = (d + 15) // 16

        @pl.loop(0, nch)
        def _chunk(g, d=d, base=base, idx_hbm=idx_hbm, tab_hbm=tab_hbm,
                   out_hbm=out_hbm, rows_v=rows_v, pk_v=pk_v, nvr=nvr):
            r0 = base + g * CH
            pltpu.sync_copy(idx_hbm.at[pl.ds(r0, CH)], idx_v)
            pltpu.async_copy(tab_hbm.at[idx_v], rows_v, sem).wait()

            @pl.loop(0, CH)
            def _row(r):
                dst = r * d
                for k in range(nvr):
                    v = rows_v[r, pl.ds(k * 16, 16)]
                    pk_v[pl.ds(dst + k * 16, 16)] = v

            pltpu.sync_copy(pk_v.at[pl.ds(0, CH * d)],
                            out_hbm.at[pl.ds(r0 * d, CH * d)])


@jax.jit
def _sc_gather(p, q, c, pPos, pNer, qPos, pQRel, pCRel,
               W_word, W_pos, W_ner, W_rel):
    out_type = [
        jax.ShapeDtypeStruct((B * LP * DW,), jnp.float32),
        jax.ShapeDtypeStruct((B * LQ * DW,), jnp.float32),
        jax.ShapeDtypeStruct((B * LC * DW,), jnp.float32),
        jax.ShapeDtypeStruct((B * LP * DP,), jnp.float32),
        jax.ShapeDtypeStruct((B * LP, DN), jnp.float32),
        jax.ShapeDtypeStruct((B * LQ * DP,), jnp.float32),
        jax.ShapeDtypeStruct((B * LP * DR,), jnp.float32),
        jax.ShapeDtypeStruct((B * LP * DR,), jnp.float32),
    ]
    scratch = [
        pltpu.VMEM((CH,), jnp.int32),
        pltpu.VMEM((CH, _pad8(DW)), jnp.float32),
        pltpu.VMEM((CH, _pad8(DP)), jnp.float32),
        pltpu.VMEM((CH, _pad8(DN)), jnp.float32),
        pltpu.VMEM((CH, _pad8(DR)), jnp.float32),
        pltpu.VMEM((CH * DW + 16,), jnp.float32),
        pltpu.VMEM((CH * DP + 16,), jnp.float32),
        pltpu.VMEM((CH * DR + 16,), jnp.float32),
        pltpu.SemaphoreType.DMA,
    ]
    mesh = plsc.VectorSubcoreMesh(core_axis_name="c", subcore_axis_name="s")
    fn = pl.kernel(_sc_body, out_type=out_type, mesh=mesh,
                   scratch_types=scratch,
                   compiler_params=pltpu.CompilerParams(
                       use_tc_tiling_on_sc=False))
    return fn(p, q, c, pPos, pNer, qPos, pQRel, pCRel,
              W_word, W_pos, W_ner, W_rel)


def _padcols(w, d_to):
    d = w.shape[1]
    if d == d_to:
        return w
    return jnp.pad(w, ((0, 0), (0, d_to - d)))


def kernel(p, q, c, pPos, pNer, qPos, pQRel, pCRel,
           W_word, W_pos, W_ner, W_rel):
    flat = [a.reshape(-1).astype(jnp.int32)
            for a in (p, q, c, pPos, pNer, qPos, pQRel, pCRel)]
    outs = _sc_gather(*flat,
                      _padcols(W_word, _pad8(DW)),
                      _padcols(W_pos, _pad8(DP)),
                      _padcols(W_ner, _pad8(DN)),
                      _padcols(W_rel, _pad8(DR)))
    return (
        outs[0].reshape(B, LP, DW),
        outs[1].reshape(B, LQ, DW),
        outs[2].reshape(B, LC, DW),
        outs[3].reshape(B, LP, DP),
        outs[4].reshape(B, LP, DN),
        outs[5].reshape(B, LQ, DP),
        outs[6].reshape(B, LP, DR),
        outs[7].reshape(B, LP, DR),
    )


# speedup vs baseline: 5.0440x; 1.2577x over previous
import jax
import jax.numpy as jnp
from jax import lax
from jax.experimental import pallas as pl
from jax.experimental.pallas import tpu as pltpu
from jax.experimental.pallas import tpu_sc as plsc

B = 4096
LP = 200
LQ = 20
LC = 20
DW = 300
DP = 12
DN = 8
DR = 10

DWP = 304
DSP = 16

NC = 2
NS = 16
NW = NC * NS
CHW = 64
CHS = 128

_TASKS = (
    ("p", "W_word", 0, DW, CHW, B * LP),
    ("q", "W_word", 1, DW, CHW, B * LQ),
    ("c", "W_word", 2, DW, CHW, B * LC),
    ("pPos", "W_pos", 3, DP, CHS, B * LP),
    ("pNer", "W_ner", 4, DN, CHS, B * LP),
    ("qPos", "W_pos", 5, DP, CHS, B * LQ),
    ("pQRel", "W_rel", 6, DR, CHS, B * LP),
    ("pCRel", "W_rel", 7, DR, CHS, B * LP),
)


def _sc_body(p, q, c, pPos, pNer, qPos, pQRel, pCRel,
             W_word, W_pos, W_ner, W_rel,
             o0, o1, o2, o3, o4, o5, o6, o7,
             idx_w0, idx_w1, idx_s0, idx_s1,
             rows_w0, rows_w1, rows_s0, rows_s1,
             pk_w0, pk_w1, pk_s0, pk_s1,
             isem0, isem1, gsem0, gsem1, osem0, osem1):
    wid = lax.axis_index("s") * NC + lax.axis_index("c")
    idx_refs = {"p": p, "q": q, "c": c, "pPos": pPos, "pNer": pNer,
                "qPos": qPos, "pQRel": pQRel, "pCRel": pCRel}
    tab_refs = {"W_word": W_word, "W_pos": W_pos, "W_ner": W_ner,
                "W_rel": W_rel}
    outs = (o0, o1, o2, o3, o4, o5, o6, o7)
    isems = (isem0, isem1)
    gsems = (gsem0, gsem1)
    osems = (osem0, osem1)

    for iname, tname, oi, d, ch, nrows in _TASKS:
        idx_hbm = idx_refs[iname]
        tab_hbm = tab_refs[tname]
        out_hbm = outs[oi]
        if tname == "W_word":
            idx_bufs, rows_bufs, pk_bufs = ((idx_w0, idx_w1),
                                            (rows_w0, rows_w1),
                                            (pk_w0, pk_w1))
        else:
            idx_bufs, rows_bufs, pk_bufs = ((idx_s0, idx_s1),
                                            (rows_s0, rows_s1),
                                            (pk_s0, pk_s1))
        per_w = nrows // NW
        nch = per_w // ch
        npair = nch // 2
        base = wid * per_w
        nvr = (d + 15) // 16

        def stage_a(g, s, idx_hbm=idx_hbm, idx_bufs=idx_bufs, ch=ch,
                    base=base):
            pltpu.async_copy(idx_hbm.at[pl.ds(base + g * ch, ch)],
                             idx_bufs[s], isems[s])

        def stage_b(s, idx_hbm=idx_hbm, tab_hbm=tab_hbm,
                    idx_bufs=idx_bufs, rows_bufs=rows_bufs, ch=ch):
            pltpu.make_async_copy(idx_hbm.at[pl.ds(0, ch)], idx_bufs[s],
                                  isems[s]).wait()
            pltpu.async_copy(tab_hbm.at[idx_bufs[s]], rows_bufs[s],
                             gsems[s])

        def stage_c(g, s, pred, tab_hbm=tab_hbm, out_hbm=out_hbm,
                    idx_bufs=idx_bufs, rows_bufs=rows_bufs,
                    pk_bufs=pk_bufs, d=d, ch=ch, base=base, nvr=nvr):
            rows_v = rows_bufs[s]
            pk_v = pk_bufs[s]
            r0 = base + g * ch
            pltpu.make_async_copy(tab_hbm.at[idx_bufs[s]], rows_v,
                                  gsems[s]).wait()

            @pl.when(pred)
            def _wait_out():
                pltpu.make_async_copy(pk_v.at[pl.ds(0, ch * d)],
                                      out_hbm.at[pl.ds(0, ch * d)],
                                      osems[s]).wait()

            @pl.loop(0, ch, unroll=2)
            def _row(r):
                dst = r * d
                for k in range(nvr):
                    v = rows_v[r, pl.ds(k * 16, 16)]
                    pk_v[pl.ds(dst + k * 16, 16)] = v

            pltpu.async_copy(pk_v.at[pl.ds(0, ch * d)],
                             out_hbm.at[pl.ds(r0 * d, ch * d)], osems[s])

        stage_a(0, 0)
        stage_a(1, 1)
        stage_b(0)

        @pl.loop(0, npair)
        def _pair(pp):
            g0 = pp * 2
            stage_b(1)
            stage_c(g0, 0, pred=pp > 0)

            @pl.when(pp + 1 < npair)
            def _pre0():
                stage_a(g0 + 2, 0)
                stage_b(0)
                stage_a(g0 + 3, 1)

            stage_c(g0 + 1, 1, pred=pp > 0)

        for s in (0, 1):
            pltpu.make_async_copy(pk_bufs[s].at[pl.ds(0, ch * d)],
                                  out_hbm.at[pl.ds(0, ch * d)],
                                  osems[s]).wait()


@jax.jit
def _sc_gather(p, q, c, pPos, pNer, qPos, pQRel, pCRel,
               W_word, W_pos, W_ner, W_rel):
    out_type = [
        jax.ShapeDtypeStruct((B * LP * DW,), jnp.float32),
        jax.ShapeDtypeStruct((B * LQ * DW,), jnp.float32),
        jax.ShapeDtypeStruct((B * LC * DW,), jnp.float32),
        jax.ShapeDtypeStruct((B * LP * DP,), jnp.float32),
        jax.ShapeDtypeStruct((B * LP * DN,), jnp.float32),
        jax.ShapeDtypeStruct((B * LQ * DP,), jnp.float32),
        jax.ShapeDtypeStruct((B * LP * DR,), jnp.float32),
        jax.ShapeDtypeStruct((B * LP * DR,), jnp.float32),
    ]
    scratch = [
        pltpu.VMEM((CHW,), jnp.int32), pltpu.VMEM((CHW,), jnp.int32),
        pltpu.VMEM((CHS,), jnp.int32), pltpu.VMEM((CHS,), jnp.int32),
        pltpu.VMEM((CHW, DWP), jnp.float32),
        pltpu.VMEM((CHW, DWP), jnp.float32),
        pltpu.VMEM((CHS, DSP), jnp.float32),
        pltpu.VMEM((CHS, DSP), jnp.float32),
        pltpu.VMEM((CHW * DW + 16,), jnp.float32),
        pltpu.VMEM((CHW * DW + 16,), jnp.float32),
        pltpu.VMEM((CHS * DP + 16,), jnp.float32),
        pltpu.VMEM((CHS * DP + 16,), jnp.float32),
        pltpu.SemaphoreType.DMA, pltpu.SemaphoreType.DMA,
        pltpu.SemaphoreType.DMA, pltpu.SemaphoreType.DMA,
        pltpu.SemaphoreType.DMA, pltpu.SemaphoreType.DMA,
    ]
    mesh = plsc.VectorSubcoreMesh(core_axis_name="c", subcore_axis_name="s")
    fn = pl.kernel(_sc_body, out_type=out_type, mesh=mesh,
                   scratch_types=scratch,
                   compiler_params=pltpu.CompilerParams(
                       use_tc_tiling_on_sc=False))
    return fn(p, q, c, pPos, pNer, qPos, pQRel, pCRel,
              W_word, W_pos, W_ner, W_rel)


def _padcols(w, d_to):
    d = w.shape[1]
    if d == d_to:
        return w
    return jnp.pad(w, ((0, 0), (0, d_to - d)))


def kernel(p, q, c, pPos, pNer, qPos, pQRel, pCRel,
           W_word, W_pos, W_ner, W_rel):
    flat = [a.reshape(-1).astype(jnp.int32)
            for a in (p, q, c, pPos, pNer, qPos, pQRel, pCRel)]
    outs = _sc_gather(*flat,
                      _padcols(W_word, DWP),
                      _padcols(W_pos, DSP),
                      _padcols(W_ner, DSP),
                      _padcols(W_rel, DSP))
    return (
        outs[0].reshape(B, LP, DW),
        outs[1].reshape(B, LQ, DW),
        outs[2].reshape(B, LC, DW),
        outs[3].reshape(B, LP, DP),
        outs[4].reshape(B, LP, DN),
        outs[5].reshape(B, LQ, DP),
        outs[6].reshape(B, LP, DR),
        outs[7].reshape(B, LP, DR),
    )


# speedup vs baseline: 5.0560x; 1.0024x over previous
import jax
import jax.numpy as jnp
from jax import lax
from jax.experimental import pallas as pl
from jax.experimental.pallas import tpu as pltpu
from jax.experimental.pallas import tpu_sc as plsc

B = 4096
LP = 200
LQ = 20
LC = 20
DW = 300
DP = 12
DN = 8
DR = 10

DWP = 304
DSP = 16

NC = 2
NS = 16
NW = NC * NS
NSLOT = 4
CHW = 32
CHS = 128

_TASKS = (
    ("p", "W_word", 0, DW, CHW, B * LP),
    ("q", "W_word", 1, DW, CHW, B * LQ),
    ("c", "W_word", 2, DW, CHW, B * LC),
    ("pPos", "W_pos", 3, DP, CHS, B * LP),
    ("pNer", "W_ner", 4, DN, CHS, B * LP),
    ("qPos", "W_pos", 5, DP, CHS, B * LQ),
    ("pQRel", "W_rel", 6, DR, CHS, B * LP),
    ("pCRel", "W_rel", 7, DR, CHS, B * LP),
)


def _sc_body(p, q, c, pPos, pNer, qPos, pQRel, pCRel,
             W_word, W_pos, W_ner, W_rel,
             o0, o1, o2, o3, o4, o5, o6, o7,
             *bufs):
    idx_w = bufs[0:4]
    idx_s = bufs[4:8]
    rows_w = bufs[8:12]
    rows_s = bufs[12:16]
    pk_w = bufs[16:20]
    pk_s = bufs[20:24]
    isems = bufs[24:28]
    gsems = bufs[28:32]
    osems = bufs[32:36]

    wid = lax.axis_index("s") * NC + lax.axis_index("c")
    idx_refs = {"p": p, "q": q, "c": c, "pPos": pPos, "pNer": pNer,
                "qPos": qPos, "pQRel": pQRel, "pCRel": pCRel}
    tab_refs = {"W_word": W_word, "W_pos": W_pos, "W_ner": W_ner,
                "W_rel": W_rel}
    outs = (o0, o1, o2, o3, o4, o5, o6, o7)

    for iname, tname, oi, d, ch, nrows in _TASKS:
        idx_hbm = idx_refs[iname]
        tab_hbm = tab_refs[tname]
        out_hbm = outs[oi]
        if tname == "W_word":
            idx_bufs, rows_bufs, pk_bufs = idx_w, rows_w, pk_w
        else:
            idx_bufs, rows_bufs, pk_bufs = idx_s, rows_s, pk_s
        per_w = nrows // NW
        nch = per_w // ch
        nq = nch // NSLOT
        base = wid * per_w
        nvr = (d + 15) // 16

        def stage_a(g, s, idx_hbm=idx_hbm, idx_bufs=idx_bufs, ch=ch,
                    base=base):
            pltpu.async_copy(idx_hbm.at[pl.ds(base + g * ch, ch)],
                             idx_bufs[s], isems[s])

        def stage_b(s, idx_hbm=idx_hbm, tab_hbm=tab_hbm,
                    idx_bufs=idx_bufs, rows_bufs=rows_bufs, ch=ch):
            pltpu.make_async_copy(idx_hbm.at[pl.ds(0, ch)], idx_bufs[s],
                                  isems[s]).wait()
            pltpu.async_copy(tab_hbm.at[idx_bufs[s]], rows_bufs[s],
                             gsems[s])

        def stage_c(g, s, pred, tab_hbm=tab_hbm, out_hbm=out_hbm,
                    idx_bufs=idx_bufs, rows_bufs=rows_bufs,
                    pk_bufs=pk_bufs, d=d, ch=ch, base=base, nvr=nvr):
            rows_v = rows_bufs[s]
            pk_v = pk_bufs[s]
            r0 = base + g * ch
            pltpu.make_async_copy(tab_hbm.at[idx_bufs[s]], rows_v,
                                  gsems[s]).wait()

            @pl.when(pred)
            def _wait_out():
                pltpu.make_async_copy(pk_v.at[pl.ds(0, ch * d)],
                                      out_hbm.at[pl.ds(0, ch * d)],
                                      osems[s]).wait()

            @pl.loop(0, ch, unroll=2)
            def _row(r):
                dst = r * d
                for k in range(nvr):
                    v = rows_v[r, pl.ds(k * 16, 16)]
                    pk_v[pl.ds(dst + k * 16, 16)] = v

            pltpu.async_copy(pk_v.at[pl.ds(0, ch * d)],
                             out_hbm.at[pl.ds(r0 * d, ch * d)], osems[s])

        for s in range(NSLOT):
            stage_a(s, s)
        for s in range(NSLOT - 1):
            stage_b(s)

        @pl.loop(0, nq)
        def _quad(t):
            g = t * NSLOT
            stage_b(NSLOT - 1)
            cond = t + 1 < nq
            for j in range(NSLOT):
                stage_c(g + j, j, pred=t > 0)

                @pl.when(cond)
                def _pre(g=g, j=j):
                    stage_a(g + NSLOT + j, j)
                    if j > 0:
                        stage_b(j - 1)

        for s in range(NSLOT):
            pltpu.make_async_copy(pk_bufs[s].at[pl.ds(0, ch * d)],
                                  out_hbm.at[pl.ds(0, ch * d)],
                                  osems[s]).wait()


@jax.jit
def _sc_gather(p, q, c, pPos, pNer, qPos, pQRel, pCRel,
               W_word, W_pos, W_ner, W_rel):
    out_type = [
        jax.ShapeDtypeStruct((B * LP * DW,), jnp.float32),
        jax.ShapeDtypeStruct((B * LQ * DW,), jnp.float32),
        jax.ShapeDtypeStruct((B * LC * DW,), jnp.float32),
        jax.ShapeDtypeStruct((B * LP * DP,), jnp.float32),
        jax.ShapeDtypeStruct((B * LP * DN,), jnp.float32),
        jax.ShapeDtypeStruct((B * LQ * DP,), jnp.float32),
        jax.ShapeDtypeStruct((B * LP * DR,), jnp.float32),
        jax.ShapeDtypeStruct((B * LP * DR,), jnp.float32),
    ]
    scratch = (
        [pltpu.VMEM((CHW,), jnp.int32)] * NSLOT
        + [pltpu.VMEM((CHS,), jnp.int32)] * NSLOT
        + [pltpu.VMEM((CHW, DWP), jnp.float32)] * NSLOT
        + [pltpu.VMEM((CHS, DSP), jnp.float32)] * NSLOT
        + [pltpu.VMEM((CHW * DW + 16,), jnp.float32)] * NSLOT
        + [pltpu.VMEM((CHS * DP + 16,), jnp.float32)] * NSLOT
        + [pltpu.SemaphoreType.DMA] * (3 * NSLOT)
    )
    mesh = plsc.VectorSubcoreMesh(core_axis_name="c", subcore_axis_name="s")
    fn = pl.kernel(_sc_body, out_type=out_type, mesh=mesh,
                   scratch_types=scratch,
                   compiler_params=pltpu.CompilerParams(
                       use_tc_tiling_on_sc=False))
    return fn(p, q, c, pPos, pNer, qPos, pQRel, pCRel,
              W_word, W_pos, W_ner, W_rel)


def _padcols(w, d_to):
    d = w.shape[1]
    if d == d_to:
        return w
    return jnp.pad(w, ((0, 0), (0, d_to - d)))


def kernel(p, q, c, pPos, pNer, qPos, pQRel, pCRel,
           W_word, W_pos, W_ner, W_rel):
    flat = [a.reshape(-1).astype(jnp.int32)
            for a in (p, q, c, pPos, pNer, qPos, pQRel, pCRel)]
    outs = _sc_gather(*flat,
                      _padcols(W_word, DWP),
                      _padcols(W_pos, DSP),
                      _padcols(W_ner, DSP),
                      _padcols(W_rel, DSP))
    return (
        outs[0].reshape(B, LP, DW),
        outs[1].reshape(B, LQ, DW),
        outs[2].reshape(B, LC, DW),
        outs[3].reshape(B, LP, DP),
        outs[4].reshape(B, LP, DN),
        outs[5].reshape(B, LQ, DP),
        outs[6].reshape(B, LP, DR),
        outs[7].reshape(B, LP, DR),
    )
